# baseline (speedup 1.0000x reference)
import functools

import jax
import jax.numpy as jnp
from jax import lax
from jax.experimental import pallas as pl
from jax.experimental.pallas import tpu as pltpu
from jax.experimental.pallas import tpu_sc as plsc

_N_CODES = 8192
_DIM = 32
_TM = 1024
_NC = 4096


def _encode_body(x_ref, emb_ref, idx_ref, esq_ref):
    @pl.when(pl.program_id(0) == 0)
    def _():
        emb = emb_ref[...]
        esq_ref[...] = jnp.sum(emb * emb, axis=0, keepdims=True)

    x = x_ref[...]
    xsq = jnp.sum(x * x, axis=1, keepdims=True)
    x2 = 2.0 * x
    tm = x.shape[0]
    colf = lax.broadcasted_iota(jnp.int32, (1, _NC), 1).astype(jnp.float32)

    ms, ifs = [], []
    for c in range(_N_CODES // _NC):
        e_chunk = emb_ref[:, pl.ds(c * _NC, _NC)]
        b_chunk = esq_ref[:, pl.ds(c * _NC, _NC)]
        sim2 = lax.dot_general(x2, e_chunk, (((1,), (0,)), ((), ())),
                               preferred_element_type=jnp.float32)
        d = (xsq + b_chunk) - sim2
        m = jnp.min(d, axis=1, keepdims=True)
        colf_c = colf + jnp.float32(c * _NC)
        i_f = jnp.min(jnp.where(d == m, colf_c, jnp.float32(3.0e38)),
                      axis=1, keepdims=True)
        ms.append(m)
        ifs.append(i_f)
    m_stack = jnp.concatenate(ms, axis=1)
    i_stack = jnp.concatenate(ifs, axis=1)
    m = jnp.min(m_stack, axis=1, keepdims=True)
    i_f = jnp.min(jnp.where(m_stack == m, i_stack, jnp.float32(3.0e38)),
                  axis=1, keepdims=True)
    idx_ref[...] = i_f.astype(jnp.int32)


def _encode(flat, embeddings):
    nb = flat.shape[0] // _TM
    return pl.pallas_call(
        _encode_body,
        grid=(nb,),
        in_specs=[pl.BlockSpec((_TM, _DIM), lambda i: (i, 0)),
                  pl.BlockSpec((_DIM, _N_CODES), lambda i: (0, 0))],
        out_specs=pl.BlockSpec((_TM, 1), lambda i: (i, 0)),
        out_shape=jax.ShapeDtypeStruct((flat.shape[0], 1), jnp.int32),
        scratch_shapes=[pltpu.VMEM((1, _N_CODES), jnp.float32)],
    )(flat, embeddings)


def _gather_sc(table, idx):
    B = idx.shape[0]
    NW = 32
    bpw = B // NW
    mesh = plsc.VectorSubcoreMesh(core_axis_name="c", subcore_axis_name="s")

    @functools.partial(
        pl.kernel, mesh=mesh,
        compiler_params=pltpu.CompilerParams(use_tc_tiling_on_sc=False),
        out_type=jax.ShapeDtypeStruct((B, _DIM), jnp.float32),
        scratch_types=[pltpu.VMEM((bpw,), jnp.int32),
                       pltpu.VMEM((bpw, _DIM), jnp.float32),
                       pltpu.SemaphoreType.DMA],
    )
    def k(table_hbm, idx_hbm, out_hbm, idx_v, rows_v, sem):
        wid = lax.axis_index("s") * 2 + lax.axis_index("c")
        base = wid * bpw
        pltpu.sync_copy(idx_hbm.at[pl.ds(base, bpw)], idx_v)
        for j in range(bpw // 128):
            pltpu.async_copy(
                table_hbm.at[idx_v.at[pl.ds(j * 128, 128)]],
                rows_v.at[pl.ds(j * 128, 128)], sem).wait()
        pltpu.sync_copy(rows_v, out_hbm.at[pl.ds(base, bpw)])

    return k(table, idx)


def kernel(x, embeddings):
    flat = x.reshape(-1, _DIM)
    idx = _encode(flat, embeddings)
    q = _gather_sc(embeddings.T, idx.reshape(-1))
    return q.reshape(x.shape)


# speedup vs baseline: 1.1202x; 1.1202x over previous
import functools

import jax
import jax.numpy as jnp
from jax import lax
from jax.experimental import pallas as pl
from jax.experimental.pallas import tpu as pltpu
from jax.experimental.pallas import tpu_sc as plsc

_N_CODES = 8192
_DIM = 32
_TM = 1024
_NC = 1024


def _encode_body(x_ref, emb_ref, idx_ref, esq_ref):
    @pl.when(pl.program_id(0) == 0)
    def _():
        emb = emb_ref[...]
        esq_ref[...] = jnp.sum(emb * emb, axis=0, keepdims=True)

    x = x_ref[...]
    xsq = jnp.sum(x * x, axis=1, keepdims=True)
    x2 = 2.0 * x
    tm = x.shape[0]
    colf = lax.broadcasted_iota(jnp.int32, (1, _NC), 1).astype(jnp.float32)

    ms, ifs = [], []
    for c in range(_N_CODES // _NC):
        e_chunk = emb_ref[:, pl.ds(c * _NC, _NC)]
        b_chunk = esq_ref[:, pl.ds(c * _NC, _NC)]
        sim2 = lax.dot_general(x2, e_chunk, (((1,), (0,)), ((), ())),
                               preferred_element_type=jnp.float32)
        d = (xsq + b_chunk) - sim2
        m = jnp.min(d, axis=1, keepdims=True)
        colf_c = colf + jnp.float32(c * _NC)
        i_f = jnp.min(jnp.where(d == m, colf_c, jnp.float32(3.0e38)),
                      axis=1, keepdims=True)
        ms.append(m)
        ifs.append(i_f)
    m_stack = jnp.concatenate(ms, axis=1)
    i_stack = jnp.concatenate(ifs, axis=1)
    m = jnp.min(m_stack, axis=1, keepdims=True)
    i_f = jnp.min(jnp.where(m_stack == m, i_stack, jnp.float32(3.0e38)),
                  axis=1, keepdims=True)
    idx_ref[...] = i_f.astype(jnp.int32)


def _encode(flat, embeddings):
    nb = flat.shape[0] // _TM
    return pl.pallas_call(
        _encode_body,
        grid=(nb,),
        in_specs=[pl.BlockSpec((_TM, _DIM), lambda i: (i, 0)),
                  pl.BlockSpec((_DIM, _N_CODES), lambda i: (0, 0))],
        out_specs=pl.BlockSpec((_TM, 1), lambda i: (i, 0)),
        out_shape=jax.ShapeDtypeStruct((flat.shape[0], 1), jnp.int32),
        scratch_shapes=[pltpu.VMEM((1, _N_CODES), jnp.float32)],
    )(flat, embeddings)


def _gather_sc(table, idx):
    B = idx.shape[0]
    NW = 32
    bpw = B // NW
    mesh = plsc.VectorSubcoreMesh(core_axis_name="c", subcore_axis_name="s")

    @functools.partial(
        pl.kernel, mesh=mesh,
        compiler_params=pltpu.CompilerParams(use_tc_tiling_on_sc=False),
        out_type=jax.ShapeDtypeStruct((B, _DIM), jnp.float32),
        scratch_types=[pltpu.VMEM((bpw,), jnp.int32),
                       pltpu.VMEM((bpw, _DIM), jnp.float32),
                       pltpu.SemaphoreType.DMA],
    )
    def k(table_hbm, idx_hbm, out_hbm, idx_v, rows_v, sem):
        wid = lax.axis_index("s") * 2 + lax.axis_index("c")
        base = wid * bpw
        pltpu.sync_copy(idx_hbm.at[pl.ds(base, bpw)], idx_v)
        for j in range(bpw // 128):
            pltpu.async_copy(
                table_hbm.at[idx_v.at[pl.ds(j * 128, 128)]],
                rows_v.at[pl.ds(j * 128, 128)], sem).wait()
        pltpu.sync_copy(rows_v, out_hbm.at[pl.ds(base, bpw)])

    return k(table, idx)


def kernel(x, embeddings):
    flat = x.reshape(-1, _DIM)
    idx = _encode(flat, embeddings)
    q = _gather_sc(embeddings.T, idx.reshape(-1))
    return q.reshape(x.shape)


# speedup vs baseline: 1.1259x; 1.0051x over previous
import functools

import jax
import jax.numpy as jnp
from jax import lax
from jax.experimental import pallas as pl
from jax.experimental.pallas import tpu as pltpu
from jax.experimental.pallas import tpu_sc as plsc

_N_CODES = 8192
_DIM = 32
_TM = 1024
_NC = 512


def _encode_body(x_ref, emb_ref, idx_ref, esq_ref):
    @pl.when(pl.program_id(0) == 0)
    def _():
        emb = emb_ref[...]
        esq_ref[...] = jnp.sum(emb * emb, axis=0, keepdims=True)

    x = x_ref[...]
    xsq = jnp.sum(x * x, axis=1, keepdims=True)
    x2 = 2.0 * x
    tm = x.shape[0]
    colf = lax.broadcasted_iota(jnp.int32, (1, _NC), 1).astype(jnp.float32)

    ms, ifs = [], []
    for c in range(_N_CODES // _NC):
        e_chunk = emb_ref[:, pl.ds(c * _NC, _NC)]
        b_chunk = esq_ref[:, pl.ds(c * _NC, _NC)]
        sim2 = lax.dot_general(x2, e_chunk, (((1,), (0,)), ((), ())),
                               preferred_element_type=jnp.float32)
        d = (xsq + b_chunk) - sim2
        m = jnp.min(d, axis=1, keepdims=True)
        colf_c = colf + jnp.float32(c * _NC)
        i_f = jnp.min(jnp.where(d == m, colf_c, jnp.float32(3.0e38)),
                      axis=1, keepdims=True)
        ms.append(m)
        ifs.append(i_f)
    m_stack = jnp.concatenate(ms, axis=1)
    i_stack = jnp.concatenate(ifs, axis=1)
    m = jnp.min(m_stack, axis=1, keepdims=True)
    i_f = jnp.min(jnp.where(m_stack == m, i_stack, jnp.float32(3.0e38)),
                  axis=1, keepdims=True)
    idx_ref[...] = i_f.astype(jnp.int32)


def _encode(flat, embeddings):
    nb = flat.shape[0] // _TM
    return pl.pallas_call(
        _encode_body,
        grid=(nb,),
        in_specs=[pl.BlockSpec((_TM, _DIM), lambda i: (i, 0)),
                  pl.BlockSpec((_DIM, _N_CODES), lambda i: (0, 0))],
        out_specs=pl.BlockSpec((_TM, 1), lambda i: (i, 0)),
        out_shape=jax.ShapeDtypeStruct((flat.shape[0], 1), jnp.int32),
        scratch_shapes=[pltpu.VMEM((1, _N_CODES), jnp.float32)],
    )(flat, embeddings)


def _gather_sc(table, idx):
    B = idx.shape[0]
    NW = 32
    bpw = B // NW
    mesh = plsc.VectorSubcoreMesh(core_axis_name="c", subcore_axis_name="s")

    @functools.partial(
        pl.kernel, mesh=mesh,
        compiler_params=pltpu.CompilerParams(use_tc_tiling_on_sc=False),
        out_type=jax.ShapeDtypeStruct((B, _DIM), jnp.float32),
        scratch_types=[pltpu.VMEM((bpw,), jnp.int32),
                       pltpu.VMEM((bpw, _DIM), jnp.float32),
                       pltpu.SemaphoreType.DMA],
    )
    def k(table_hbm, idx_hbm, out_hbm, idx_v, rows_v, sem):
        wid = lax.axis_index("s") * 2 + lax.axis_index("c")
        base = wid * bpw
        pltpu.sync_copy(idx_hbm.at[pl.ds(base, bpw)], idx_v)
        for j in range(bpw // 128):
            pltpu.async_copy(
                table_hbm.at[idx_v.at[pl.ds(j * 128, 128)]],
                rows_v.at[pl.ds(j * 128, 128)], sem).wait()
        pltpu.sync_copy(rows_v, out_hbm.at[pl.ds(base, bpw)])

    return k(table, idx)


def kernel(x, embeddings):
    flat = x.reshape(-1, _DIM)
    idx = _encode(flat, embeddings)
    q = _gather_sc(embeddings.T, idx.reshape(-1))
    return q.reshape(x.shape)
